# baseline (speedup 1.0000x reference)
import functools

import jax
import jax.numpy as jnp
from jax import lax
from jax.experimental import pallas as pl
from jax.experimental.pallas import tpu as pltpu
from jax.experimental.pallas import tpu_sc as plsc

N_NODES = 10000
N_PAD = 10240
N_EDGES = 320000
NF = 128
NCLS = 40

NC, NS = 2, 16
NW = NC * NS
CH = 80
CPT = 128
SEG = 40
NSEG = CPT // SEG
E_PAD = NW * CPT * CH
RPT = N_PAD // NS
DEGW = 128
PAD_NODE = N_PAD - 8

_mesh = plsc.VectorSubcoreMesh(core_axis_name="c", subcore_axis_name="s")


@functools.partial(
    pl.kernel,
    out_type=jax.ShapeDtypeStruct((NC, N_PAD, DEGW), jnp.float32),
    mesh=_mesh,
    scratch_types=[
        pltpu.VMEM((CPT, CH), jnp.int32),
        pltpu.VMEM((CH, DEGW), jnp.float32),
        pltpu.VMEM_SHARED((N_PAD, DEGW), jnp.float32),
    ],
)
def _deg_kernel(col_hbm, ones_hbm, zeros_hbm, out_hbm, cidx, ones_v, acc_sh):
    cid = lax.axis_index("c")
    sid = lax.axis_index("s")
    wid = cid * NS + sid
    sl = pl.ds(sid * RPT, RPT)
    pltpu.sync_copy(ones_hbm, ones_v)
    pltpu.sync_copy(col_hbm.at[pl.ds(wid * CPT, CPT)], cidx)
    pltpu.sync_copy(zeros_hbm.at[sl], acc_sh.at[sl])
    plsc.subcore_barrier()

    def body(i, carry):
        pltpu.sync_copy(ones_v, acc_sh.at[cidx.at[i]], add=True)
        return carry

    lax.fori_loop(0, CPT, body, 0)
    plsc.subcore_barrier()
    pltpu.sync_copy(acc_sh.at[sl], out_hbm.at[cid, sl])


@functools.partial(
    pl.kernel,
    out_type=jax.ShapeDtypeStruct((NC, N_PAD, NF), jnp.float32),
    mesh=_mesh,
    scratch_types=[
        pltpu.VMEM((CH,), jnp.int32),
        pltpu.VMEM((CH,), jnp.int32),
        pltpu.VMEM((CH, NF), jnp.float32),
        pltpu.SemaphoreType.DMA,
        pltpu.VMEM_SHARED((N_PAD, NF), jnp.float32),
    ],
)
def _agg_kernel(g_hbm, row_hbm, col_hbm, zeros_hbm, out_hbm,
                ridx, cidx, rows, sem, acc_sh):
    cid = lax.axis_index("c")
    sid = lax.axis_index("s")
    wid = cid * NS + sid
    sl = pl.ds(sid * RPT, RPT)

    @pl.when(cid == 0)
    def _():
        pltpu.sync_copy(g_hbm.at[sl], acc_sh.at[sl])

    @pl.when(cid != 0)
    def _():
        pltpu.sync_copy(zeros_hbm.at[sl], acc_sh.at[sl])

    plsc.subcore_barrier()
    base = wid * CPT * CH

    def body(i, carry):
        off = base + i * CH
        pltpu.sync_copy(row_hbm.at[pl.ds(off, CH)], ridx)
        pltpu.sync_copy(col_hbm.at[pl.ds(off, CH)], cidx)
        pltpu.async_copy(g_hbm.at[ridx], rows, sem).wait()
        pltpu.sync_copy(rows, acc_sh.at[cidx], add=True)
        return carry

    lax.fori_loop(0, CPT, body, 0)
    plsc.subcore_barrier()
    pltpu.sync_copy(acc_sh.at[sl], out_hbm.at[cid, sl])


BLK = 1000


def _t1_body(x_ref, w_ref, dinv_ref, o_ref):
    h = jnp.dot(x_ref[...], w_ref[...], preferred_element_type=jnp.float32)
    o_ref[...] = dinv_ref[...] * h


_t1 = pl.pallas_call(
    _t1_body,
    grid=(N_NODES // BLK,),
    in_specs=[
        pl.BlockSpec((BLK, NF), lambda i: (i, 0)),
        pl.BlockSpec((NF, NF), lambda i: (0, 0)),
        pl.BlockSpec((BLK, 1), lambda i: (i, 0)),
    ],
    out_specs=pl.BlockSpec((BLK, NF), lambda i: (i, 0)),
    out_shape=jax.ShapeDtypeStruct((N_NODES, NF), jnp.float32),
)


def _t2_body(acc_a_ref, acc_b_ref, dinv_ref, b_ref, w_ref, o_ref):
    dinv = dinv_ref[...]
    h = dinv * (acc_a_ref[...] + acc_b_ref[...]) + b_ref[...]
    h = jnp.maximum(h, 0.0)
    o_ref[...] = dinv * jnp.dot(h, w_ref[...], preferred_element_type=jnp.float32)


_t2 = pl.pallas_call(
    _t2_body,
    grid=(N_NODES // BLK,),
    in_specs=[
        pl.BlockSpec((BLK, NF), lambda i: (i, 0)),
        pl.BlockSpec((BLK, NF), lambda i: (i, 0)),
        pl.BlockSpec((BLK, 1), lambda i: (i, 0)),
        pl.BlockSpec((1, NF), lambda i: (0, 0)),
        pl.BlockSpec((NF, NF), lambda i: (0, 0)),
    ],
    out_specs=pl.BlockSpec((BLK, NF), lambda i: (i, 0)),
    out_shape=jax.ShapeDtypeStruct((N_NODES, NF), jnp.float32),
)


def _t3_body(acc_a_ref, acc_b_ref, dinv_ref, b_ref, wc_ref, bc_ref, o_ref):
    h = dinv_ref[...] * (acc_a_ref[...] + acc_b_ref[...]) + b_ref[...]
    h = jnp.maximum(h, 0.0)
    o_ref[...] = jnp.dot(h, wc_ref[...], preferred_element_type=jnp.float32) + bc_ref[...]


_t3 = pl.pallas_call(
    _t3_body,
    grid=(N_NODES // BLK,),
    in_specs=[
        pl.BlockSpec((BLK, NF), lambda i: (i, 0)),
        pl.BlockSpec((BLK, NF), lambda i: (i, 0)),
        pl.BlockSpec((BLK, 1), lambda i: (i, 0)),
        pl.BlockSpec((1, NF), lambda i: (0, 0)),
        pl.BlockSpec((NF, NCLS), lambda i: (0, 0)),
        pl.BlockSpec((1, NCLS), lambda i: (0, 0)),
    ],
    out_specs=pl.BlockSpec((BLK, NCLS), lambda i: (i, 0)),
    out_shape=jax.ShapeDtypeStruct((N_NODES, NCLS), jnp.float32),
)


def _pad_nodes(a):
    return jnp.pad(a, ((0, N_PAD - N_NODES), (0, 0)))


def kernel(x, edge_index, W1, b1, W2, b2, Wc, bc):
    row = jnp.concatenate(
        [edge_index[0].astype(jnp.int32),
         jnp.zeros((E_PAD - N_EDGES,), jnp.int32)])
    col = jnp.concatenate(
        [edge_index[1].astype(jnp.int32),
         jnp.full((E_PAD - N_EDGES,), PAD_NODE, jnp.int32)])
    col2d = col.reshape(NW * CPT, CH)
    ones16 = jnp.ones((CH, DEGW), jnp.float32)
    zeros16 = jnp.zeros((N_PAD, DEGW), jnp.float32)
    zerosf = jnp.zeros((N_PAD, NF), jnp.float32)

    deg_parts = _deg_kernel(col2d, ones16, zeros16)
    deg = deg_parts[0, :N_NODES, 0] + deg_parts[1, :N_NODES, 0] + 1.0
    dinv = jax.lax.rsqrt(deg).reshape(N_NODES, 1)

    g1 = _pad_nodes(_t1(x, W1, dinv))
    acc1 = _agg_kernel(g1, row, col, zerosf)
    g2 = _pad_nodes(
        _t2(acc1[0, :N_NODES], acc1[1, :N_NODES], dinv, b1.reshape(1, NF), W2))
    acc2 = _agg_kernel(g2, row, col, zerosf)
    out = _t3(acc2[0, :N_NODES], acc2[1, :N_NODES], dinv,
              b2.reshape(1, NF), Wc, bc.reshape(1, NCLS))
    return out


# speedup vs baseline: 1.0106x; 1.0106x over previous
import functools

import jax
import jax.numpy as jnp
from jax import lax
from jax.experimental import pallas as pl
from jax.experimental.pallas import tpu as pltpu
from jax.experimental.pallas import tpu_sc as plsc

N_NODES = 10000
N_PAD = 10240
N_EDGES = 320000
NF = 128
NCLS = 40

NC, NS = 2, 16
NW = NC * NS
CH = 80
CPT = 128
SEG = 40
NSEG = CPT // SEG
E_PAD = NW * CPT * CH
RPT = N_PAD // NS
DEGW = 128
DEG_CH = 128
DEG_CPT = E_PAD // (NW * DEG_CH)
PAD_NODE = N_PAD - 8

_mesh = plsc.VectorSubcoreMesh(core_axis_name="c", subcore_axis_name="s")


@functools.partial(
    pl.kernel,
    out_type=jax.ShapeDtypeStruct((NC, N_PAD, DEGW), jnp.float32),
    mesh=_mesh,
    scratch_types=[
        pltpu.VMEM((DEG_CPT, DEG_CH), jnp.int32),
        pltpu.VMEM((DEG_CH, DEGW), jnp.float32),
        pltpu.VMEM_SHARED((N_PAD, DEGW), jnp.float32),
    ],
)
def _deg_kernel(col_hbm, ones_hbm, zeros_hbm, out_hbm, cidx, ones_v, acc_sh):
    cid = lax.axis_index("c")
    sid = lax.axis_index("s")
    wid = cid * NS + sid
    sl = pl.ds(sid * RPT, RPT)
    pltpu.sync_copy(ones_hbm, ones_v)
    pltpu.sync_copy(col_hbm.at[pl.ds(wid * DEG_CPT, DEG_CPT)], cidx)
    pltpu.sync_copy(zeros_hbm.at[sl], acc_sh.at[sl])
    plsc.subcore_barrier()

    def body(i, carry):
        pltpu.sync_copy(ones_v, acc_sh.at[cidx.at[i]], add=True)
        return carry

    lax.fori_loop(0, DEG_CPT, body, 0)
    plsc.subcore_barrier()
    pltpu.sync_copy(acc_sh.at[sl], out_hbm.at[cid, sl])


@functools.partial(
    pl.kernel,
    out_type=jax.ShapeDtypeStruct((NC, N_PAD, NF), jnp.float32),
    mesh=_mesh,
    scratch_types=[
        pltpu.VMEM((CH,), jnp.int32),
        pltpu.VMEM((CH,), jnp.int32),
        pltpu.VMEM((CH, NF), jnp.float32),
        pltpu.SemaphoreType.DMA,
        pltpu.VMEM_SHARED((N_PAD, NF), jnp.float32),
    ],
)
def _agg_kernel(g_hbm, row_hbm, col_hbm, zeros_hbm, out_hbm,
                ridx, cidx, rows, sem, acc_sh):
    cid = lax.axis_index("c")
    sid = lax.axis_index("s")
    wid = cid * NS + sid
    sl = pl.ds(sid * RPT, RPT)

    @pl.when(cid == 0)
    def _():
        pltpu.sync_copy(g_hbm.at[sl], acc_sh.at[sl])

    @pl.when(cid != 0)
    def _():
        pltpu.sync_copy(zeros_hbm.at[sl], acc_sh.at[sl])

    plsc.subcore_barrier()
    base = wid * CPT * CH

    def body(i, carry):
        off = base + i * CH
        pltpu.sync_copy(row_hbm.at[pl.ds(off, CH)], ridx)
        pltpu.sync_copy(col_hbm.at[pl.ds(off, CH)], cidx)
        pltpu.async_copy(g_hbm.at[ridx], rows, sem).wait()
        pltpu.sync_copy(rows, acc_sh.at[cidx], add=True)
        return carry

    lax.fori_loop(0, CPT, body, 0)
    plsc.subcore_barrier()
    pltpu.sync_copy(acc_sh.at[sl], out_hbm.at[cid, sl])


BLK = 1000


def _t1_body(x_ref, w_ref, dinv_ref, o_ref):
    h = jnp.dot(x_ref[...], w_ref[...], preferred_element_type=jnp.float32)
    o_ref[...] = dinv_ref[...] * h


_t1 = pl.pallas_call(
    _t1_body,
    grid=(N_NODES // BLK,),
    in_specs=[
        pl.BlockSpec((BLK, NF), lambda i: (i, 0)),
        pl.BlockSpec((NF, NF), lambda i: (0, 0)),
        pl.BlockSpec((BLK, 1), lambda i: (i, 0)),
    ],
    out_specs=pl.BlockSpec((BLK, NF), lambda i: (i, 0)),
    out_shape=jax.ShapeDtypeStruct((N_NODES, NF), jnp.float32),
)


def _t2_body(acc_a_ref, acc_b_ref, dinv_ref, b_ref, w_ref, o_ref):
    dinv = dinv_ref[...]
    h = dinv * (acc_a_ref[...] + acc_b_ref[...]) + b_ref[...]
    h = jnp.maximum(h, 0.0)
    o_ref[...] = dinv * jnp.dot(h, w_ref[...], preferred_element_type=jnp.float32)


_t2 = pl.pallas_call(
    _t2_body,
    grid=(N_NODES // BLK,),
    in_specs=[
        pl.BlockSpec((BLK, NF), lambda i: (i, 0)),
        pl.BlockSpec((BLK, NF), lambda i: (i, 0)),
        pl.BlockSpec((BLK, 1), lambda i: (i, 0)),
        pl.BlockSpec((1, NF), lambda i: (0, 0)),
        pl.BlockSpec((NF, NF), lambda i: (0, 0)),
    ],
    out_specs=pl.BlockSpec((BLK, NF), lambda i: (i, 0)),
    out_shape=jax.ShapeDtypeStruct((N_NODES, NF), jnp.float32),
)


def _t3_body(acc_a_ref, acc_b_ref, dinv_ref, b_ref, wc_ref, bc_ref, o_ref):
    h = dinv_ref[...] * (acc_a_ref[...] + acc_b_ref[...]) + b_ref[...]
    h = jnp.maximum(h, 0.0)
    o_ref[...] = jnp.dot(h, wc_ref[...], preferred_element_type=jnp.float32) + bc_ref[...]


_t3 = pl.pallas_call(
    _t3_body,
    grid=(N_NODES // BLK,),
    in_specs=[
        pl.BlockSpec((BLK, NF), lambda i: (i, 0)),
        pl.BlockSpec((BLK, NF), lambda i: (i, 0)),
        pl.BlockSpec((BLK, 1), lambda i: (i, 0)),
        pl.BlockSpec((1, NF), lambda i: (0, 0)),
        pl.BlockSpec((NF, NCLS), lambda i: (0, 0)),
        pl.BlockSpec((1, NCLS), lambda i: (0, 0)),
    ],
    out_specs=pl.BlockSpec((BLK, NCLS), lambda i: (i, 0)),
    out_shape=jax.ShapeDtypeStruct((N_NODES, NCLS), jnp.float32),
)


def _pad_nodes(a):
    return jnp.pad(a, ((0, N_PAD - N_NODES), (0, 0)))


def kernel(x, edge_index, W1, b1, W2, b2, Wc, bc):
    row = jnp.concatenate(
        [edge_index[0].astype(jnp.int32),
         jnp.zeros((E_PAD - N_EDGES,), jnp.int32)])
    col = jnp.concatenate(
        [edge_index[1].astype(jnp.int32),
         jnp.full((E_PAD - N_EDGES,), PAD_NODE, jnp.int32)])
    col2d = col.reshape(NW * DEG_CPT, DEG_CH)
    ones16 = jnp.ones((DEG_CH, DEGW), jnp.float32)
    zeros16 = jnp.zeros((N_PAD, DEGW), jnp.float32)
    zerosf = jnp.zeros((N_PAD, NF), jnp.float32)

    deg_parts = _deg_kernel(col2d, ones16, zeros16)
    deg = deg_parts[0, :N_NODES, 0] + deg_parts[1, :N_NODES, 0] + 1.0
    dinv = jax.lax.rsqrt(deg).reshape(N_NODES, 1)

    g1 = _pad_nodes(_t1(x, W1, dinv))
    acc1 = _agg_kernel(g1, row, col, zerosf)
    g2 = _pad_nodes(
        _t2(acc1[0, :N_NODES], acc1[1, :N_NODES], dinv, b1.reshape(1, NF), W2))
    acc2 = _agg_kernel(g2, row, col, zerosf)
    out = _t3(acc2[0, :N_NODES], acc2[1, :N_NODES], dinv,
              b2.reshape(1, NF), Wc, bc.reshape(1, NCLS))
    return out


# speedup vs baseline: 1.8301x; 1.8109x over previous
import functools

import jax
import jax.numpy as jnp
from jax import lax
from jax.experimental import pallas as pl
from jax.experimental.pallas import tpu as pltpu
from jax.experimental.pallas import tpu_sc as plsc

N_NODES = 10000
N_PAD = 10240
N_EDGES = 320000
NF = 128
NCLS = 40

NC, NS = 2, 16
NW = NC * NS
CH = 80
CPT = 128
SEG = 40
NSEG = CPT // SEG
E_PAD = NW * CPT * CH
RPT = N_PAD // NS
DEGW = 128
DEG_CH = 128
DEG_CPT = E_PAD // (NW * DEG_CH)
PAD_NODE = N_PAD - 8

_mesh = plsc.VectorSubcoreMesh(core_axis_name="c", subcore_axis_name="s")


@functools.partial(
    pl.kernel,
    out_type=jax.ShapeDtypeStruct((NC, N_PAD, DEGW), jnp.float32),
    mesh=_mesh,
    scratch_types=[
        pltpu.VMEM((DEG_CPT, DEG_CH), jnp.int32),
        pltpu.VMEM((DEG_CH, DEGW), jnp.float32),
        pltpu.VMEM_SHARED((N_PAD, DEGW), jnp.float32),
    ],
)
def _deg_kernel(col_hbm, ones_hbm, zeros_hbm, out_hbm, cidx, ones_v, acc_sh):
    cid = lax.axis_index("c")
    sid = lax.axis_index("s")
    wid = cid * NS + sid
    sl = pl.ds(sid * RPT, RPT)
    pltpu.sync_copy(ones_hbm, ones_v)
    pltpu.sync_copy(col_hbm.at[pl.ds(wid * DEG_CPT, DEG_CPT)], cidx)
    pltpu.sync_copy(zeros_hbm.at[sl], acc_sh.at[sl])
    plsc.subcore_barrier()

    def body(i, carry):
        pltpu.sync_copy(ones_v, acc_sh.at[cidx.at[i]], add=True)
        return carry

    lax.fori_loop(0, DEG_CPT, body, 0)
    plsc.subcore_barrier()
    pltpu.sync_copy(acc_sh.at[sl], out_hbm.at[cid, sl])


@functools.partial(
    pl.kernel,
    out_type=jax.ShapeDtypeStruct((NC, N_PAD, NF), jnp.float32),
    mesh=_mesh,
    scratch_types=[
        pltpu.VMEM((CH,), jnp.int32),
        pltpu.VMEM((CH,), jnp.int32),
        pltpu.VMEM((CH, NF), jnp.float32),
        pltpu.SemaphoreType.DMA,
        pltpu.VMEM_SHARED((N_PAD, NF), jnp.float32),
    ],
)
def _agg_kernel(g_hbm, row_hbm, col_hbm, zeros_hbm, out_hbm,
                ridx, cidx, rows, sem, acc_sh):
    cid = lax.axis_index("c")
    sid = lax.axis_index("s")
    wid = cid * NS + sid
    sl = pl.ds(sid * RPT, RPT)

    @pl.when(cid == 0)
    def _():
        pltpu.sync_copy(g_hbm.at[sl], acc_sh.at[sl])

    @pl.when(cid != 0)
    def _():
        pltpu.sync_copy(zeros_hbm.at[sl], acc_sh.at[sl])

    plsc.subcore_barrier()
    base = wid * CPT * CH

    def body(i, carry):
        off = base + i * CH
        pltpu.sync_copy(row_hbm.at[pl.ds(off, CH)], ridx)
        pltpu.sync_copy(col_hbm.at[pl.ds(off, CH)], cidx)
        pltpu.async_copy(g_hbm.at[ridx], rows, sem).wait()
        pltpu.sync_copy(rows, acc_sh.at[cidx], add=True)
        return carry

    lax.fori_loop(0, CPT, body, 0)
    plsc.subcore_barrier()
    pltpu.sync_copy(acc_sh.at[sl], out_hbm.at[cid, sl])


BLK = 1000


def _t1_body(x_ref, w_ref, dinv_ref, o_ref):
    h = jnp.dot(x_ref[...], w_ref[...], preferred_element_type=jnp.float32)
    o_ref[...] = dinv_ref[...] * h


_t1 = pl.pallas_call(
    _t1_body,
    grid=(N_NODES // BLK,),
    in_specs=[
        pl.BlockSpec((BLK, NF), lambda i: (i, 0)),
        pl.BlockSpec((NF, NF), lambda i: (0, 0)),
        pl.BlockSpec((BLK, 1), lambda i: (i, 0)),
    ],
    out_specs=pl.BlockSpec((BLK, NF), lambda i: (i, 0)),
    out_shape=jax.ShapeDtypeStruct((N_NODES, NF), jnp.float32),
)


def _t2_body(acc_a_ref, acc_b_ref, dinv_ref, b_ref, w_ref, o_ref):
    dinv = dinv_ref[...]
    h = dinv * (acc_a_ref[...] + acc_b_ref[...]) + b_ref[...]
    h = jnp.maximum(h, 0.0)
    o_ref[...] = dinv * jnp.dot(h, w_ref[...], preferred_element_type=jnp.float32)


_t2 = pl.pallas_call(
    _t2_body,
    grid=(N_NODES // BLK,),
    in_specs=[
        pl.BlockSpec((BLK, NF), lambda i: (i, 0)),
        pl.BlockSpec((BLK, NF), lambda i: (i, 0)),
        pl.BlockSpec((BLK, 1), lambda i: (i, 0)),
        pl.BlockSpec((1, NF), lambda i: (0, 0)),
        pl.BlockSpec((NF, NF), lambda i: (0, 0)),
    ],
    out_specs=pl.BlockSpec((BLK, NF), lambda i: (i, 0)),
    out_shape=jax.ShapeDtypeStruct((N_NODES, NF), jnp.float32),
)


def _t3_body(acc_a_ref, acc_b_ref, dinv_ref, b_ref, wc_ref, bc_ref, o_ref):
    h = dinv_ref[...] * (acc_a_ref[...] + acc_b_ref[...]) + b_ref[...]
    h = jnp.maximum(h, 0.0)
    o_ref[...] = jnp.dot(h, wc_ref[...], preferred_element_type=jnp.float32) + bc_ref[...]


_t3 = pl.pallas_call(
    _t3_body,
    grid=(N_NODES // BLK,),
    in_specs=[
        pl.BlockSpec((BLK, NF), lambda i: (i, 0)),
        pl.BlockSpec((BLK, NF), lambda i: (i, 0)),
        pl.BlockSpec((BLK, 1), lambda i: (i, 0)),
        pl.BlockSpec((1, NF), lambda i: (0, 0)),
        pl.BlockSpec((NF, NCLS), lambda i: (0, 0)),
        pl.BlockSpec((1, NCLS), lambda i: (0, 0)),
    ],
    out_specs=pl.BlockSpec((BLK, NCLS), lambda i: (i, 0)),
    out_shape=jax.ShapeDtypeStruct((N_NODES, NCLS), jnp.float32),
)


def _pad_nodes(a):
    return jnp.pad(a, ((0, N_PAD - N_NODES), (0, 0)))


def kernel(x, edge_index, W1, b1, W2, b2, Wc, bc):
    pad_tgt = jnp.tile(jnp.arange(N_NODES, N_PAD, dtype=jnp.int32),
                       (E_PAD - N_EDGES) // (N_PAD - N_NODES))
    row = jnp.concatenate([edge_index[0].astype(jnp.int32), pad_tgt])
    col = jnp.concatenate([edge_index[1].astype(jnp.int32), pad_tgt])
    col2d = col.reshape(NW * DEG_CPT, DEG_CH)
    ones16 = jnp.ones((DEG_CH, DEGW), jnp.float32)
    zeros16 = jnp.zeros((N_PAD, DEGW), jnp.float32)
    zerosf = jnp.zeros((N_PAD, NF), jnp.float32)

    deg_parts = _deg_kernel(col2d, ones16, zeros16)
    deg = deg_parts[0, :N_NODES, 0] + deg_parts[1, :N_NODES, 0] + 1.0
    dinv = jax.lax.rsqrt(deg).reshape(N_NODES, 1)

    g1 = _pad_nodes(_t1(x, W1, dinv))
    acc1 = _agg_kernel(g1, row, col, zerosf)
    g2 = _pad_nodes(
        _t2(acc1[0, :N_NODES], acc1[1, :N_NODES], dinv, b1.reshape(1, NF), W2))
    acc2 = _agg_kernel(g2, row, col, zerosf)
    out = _t3(acc2[0, :N_NODES], acc2[1, :N_NODES], dinv,
              b2.reshape(1, NF), Wc, bc.reshape(1, NCLS))
    return out


# speedup vs baseline: 2.7651x; 1.5109x over previous
import functools

import jax
import jax.numpy as jnp
from jax import lax
from jax.experimental import pallas as pl
from jax.experimental.pallas import tpu as pltpu
from jax.experimental.pallas import tpu_sc as plsc

N_NODES = 10000
N_PAD = 10240
N_EDGES = 320000
NF = 128
NCLS = 40

NC, NS = 2, 16
NW = NC * NS
CH = 128
CPT = 80
SEG = 40
NSEG = CPT // SEG
E_PAD = NW * CPT * CH
RPT = N_PAD // NS
DEGW = 128
DEG_CH = 128
DEG_CPT = E_PAD // (NW * DEG_CH)
PAD_NODE = N_PAD - 8

_mesh = plsc.VectorSubcoreMesh(core_axis_name="c", subcore_axis_name="s")


@functools.partial(
    pl.kernel,
    out_type=jax.ShapeDtypeStruct((NC, N_PAD, DEGW), jnp.float32),
    mesh=_mesh,
    scratch_types=[
        pltpu.VMEM((DEG_CPT, DEG_CH), jnp.int32),
        pltpu.VMEM((DEG_CH, DEGW), jnp.float32),
        pltpu.VMEM_SHARED((N_PAD, DEGW), jnp.float32),
    ],
)
def _deg_kernel(col_hbm, ones_hbm, zeros_hbm, out_hbm, cidx, ones_v, acc_sh):
    cid = lax.axis_index("c")
    sid = lax.axis_index("s")
    wid = cid * NS + sid
    sl = pl.ds(sid * RPT, RPT)
    pltpu.sync_copy(ones_hbm, ones_v)
    pltpu.sync_copy(col_hbm.at[pl.ds(wid * DEG_CPT, DEG_CPT)], cidx)
    pltpu.sync_copy(zeros_hbm.at[sl], acc_sh.at[sl])
    plsc.subcore_barrier()

    def body(i, carry):
        pltpu.sync_copy(ones_v, acc_sh.at[cidx.at[i]], add=True)
        return carry

    lax.fori_loop(0, DEG_CPT, body, 0)
    plsc.subcore_barrier()
    pltpu.sync_copy(acc_sh.at[sl], out_hbm.at[cid, sl])


@functools.partial(
    pl.kernel,
    out_type=jax.ShapeDtypeStruct((NC, N_PAD, NF), jnp.float32),
    mesh=_mesh,
    scratch_types=[
        pltpu.VMEM((SEG, CH), jnp.int32),
        pltpu.VMEM((SEG, CH), jnp.int32),
        pltpu.VMEM((CH, NF), jnp.float32),
        pltpu.SemaphoreType.DMA,
        pltpu.VMEM_SHARED((N_PAD, NF), jnp.float32),
    ],
)
def _agg_kernel(g_hbm, row_hbm, col_hbm, zeros_hbm, out_hbm,
                ridx, cidx, rows, sem, acc_sh):
    cid = lax.axis_index("c")
    sid = lax.axis_index("s")
    wid = cid * NS + sid
    sl = pl.ds(sid * RPT, RPT)

    @pl.when(cid == 0)
    def _():
        pltpu.sync_copy(g_hbm.at[sl], acc_sh.at[sl])

    @pl.when(cid != 0)
    def _():
        pltpu.sync_copy(zeros_hbm.at[sl], acc_sh.at[sl])

    plsc.subcore_barrier()

    for s in range(NSEG):
        b = wid * CPT + s * SEG
        pltpu.sync_copy(row_hbm.at[pl.ds(b, SEG)], ridx)
        pltpu.sync_copy(col_hbm.at[pl.ds(b, SEG)], cidx)

        def body(i, carry):
            pltpu.async_copy(g_hbm.at[ridx.at[i]], rows, sem).wait()
            pltpu.sync_copy(rows, acc_sh.at[cidx.at[i]], add=True)
            return carry

        lax.fori_loop(0, SEG, body, 0)

    plsc.subcore_barrier()
    pltpu.sync_copy(acc_sh.at[sl], out_hbm.at[cid, sl])


BLK = 1000


def _t1_body(x_ref, w_ref, dinv_ref, o_ref):
    h = jnp.dot(x_ref[...], w_ref[...], preferred_element_type=jnp.float32)
    o_ref[...] = dinv_ref[...] * h


_t1 = pl.pallas_call(
    _t1_body,
    grid=(N_NODES // BLK,),
    in_specs=[
        pl.BlockSpec((BLK, NF), lambda i: (i, 0)),
        pl.BlockSpec((NF, NF), lambda i: (0, 0)),
        pl.BlockSpec((BLK, 1), lambda i: (i, 0)),
    ],
    out_specs=pl.BlockSpec((BLK, NF), lambda i: (i, 0)),
    out_shape=jax.ShapeDtypeStruct((N_NODES, NF), jnp.float32),
)


def _t2_body(acc_a_ref, acc_b_ref, dinv_ref, b_ref, w_ref, o_ref):
    dinv = dinv_ref[...]
    h = dinv * (acc_a_ref[...] + acc_b_ref[...]) + b_ref[...]
    h = jnp.maximum(h, 0.0)
    o_ref[...] = dinv * jnp.dot(h, w_ref[...], preferred_element_type=jnp.float32)


_t2 = pl.pallas_call(
    _t2_body,
    grid=(N_NODES // BLK,),
    in_specs=[
        pl.BlockSpec((BLK, NF), lambda i: (i, 0)),
        pl.BlockSpec((BLK, NF), lambda i: (i, 0)),
        pl.BlockSpec((BLK, 1), lambda i: (i, 0)),
        pl.BlockSpec((1, NF), lambda i: (0, 0)),
        pl.BlockSpec((NF, NF), lambda i: (0, 0)),
    ],
    out_specs=pl.BlockSpec((BLK, NF), lambda i: (i, 0)),
    out_shape=jax.ShapeDtypeStruct((N_NODES, NF), jnp.float32),
)


def _t3_body(acc_a_ref, acc_b_ref, dinv_ref, b_ref, wc_ref, bc_ref, o_ref):
    h = dinv_ref[...] * (acc_a_ref[...] + acc_b_ref[...]) + b_ref[...]
    h = jnp.maximum(h, 0.0)
    o_ref[...] = jnp.dot(h, wc_ref[...], preferred_element_type=jnp.float32) + bc_ref[...]


_t3 = pl.pallas_call(
    _t3_body,
    grid=(N_NODES // BLK,),
    in_specs=[
        pl.BlockSpec((BLK, NF), lambda i: (i, 0)),
        pl.BlockSpec((BLK, NF), lambda i: (i, 0)),
        pl.BlockSpec((BLK, 1), lambda i: (i, 0)),
        pl.BlockSpec((1, NF), lambda i: (0, 0)),
        pl.BlockSpec((NF, NCLS), lambda i: (0, 0)),
        pl.BlockSpec((1, NCLS), lambda i: (0, 0)),
    ],
    out_specs=pl.BlockSpec((BLK, NCLS), lambda i: (i, 0)),
    out_shape=jax.ShapeDtypeStruct((N_NODES, NCLS), jnp.float32),
)


def _pad_nodes(a):
    return jnp.pad(a, ((0, N_PAD - N_NODES), (0, 0)))


def kernel(x, edge_index, W1, b1, W2, b2, Wc, bc):
    pad_tgt = jnp.tile(jnp.arange(N_NODES, N_PAD, dtype=jnp.int32),
                       (E_PAD - N_EDGES) // (N_PAD - N_NODES))
    row = jnp.concatenate([edge_index[0].astype(jnp.int32), pad_tgt])
    col = jnp.concatenate([edge_index[1].astype(jnp.int32), pad_tgt])
    row2d = row.reshape(NW * DEG_CPT, DEG_CH)
    col2d = col.reshape(NW * DEG_CPT, DEG_CH)
    ones16 = jnp.ones((DEG_CH, DEGW), jnp.float32)
    zeros16 = jnp.zeros((N_PAD, DEGW), jnp.float32)
    zerosf = jnp.zeros((N_PAD, NF), jnp.float32)

    deg_parts = _deg_kernel(col2d, ones16, zeros16)
    deg = deg_parts[0, :N_NODES, 0] + deg_parts[1, :N_NODES, 0] + 1.0
    dinv = jax.lax.rsqrt(deg).reshape(N_NODES, 1)

    g1 = _pad_nodes(_t1(x, W1, dinv))
    acc1 = _agg_kernel(g1, row2d, col2d, zerosf)
    g2 = _pad_nodes(
        _t2(acc1[0, :N_NODES], acc1[1, :N_NODES], dinv, b1.reshape(1, NF), W2))
    acc2 = _agg_kernel(g2, row2d, col2d, zerosf)
    out = _t3(acc2[0, :N_NODES], acc2[1, :N_NODES], dinv,
              b2.reshape(1, NF), Wc, bc.reshape(1, NCLS))
    return out


# speedup vs baseline: 3.0558x; 1.1052x over previous
import functools

import jax
import jax.numpy as jnp
from jax import lax
from jax.experimental import pallas as pl
from jax.experimental.pallas import tpu as pltpu
from jax.experimental.pallas import tpu_sc as plsc

N_NODES = 10000
N_PAD = 10240
N_EDGES = 320000
NF = 128
NCLS = 40

NC, NS = 2, 16
NW = NC * NS
CH = 128
CPT = 80
SEG = 40
NSEG = CPT // SEG
E_PAD = NW * CPT * CH
RPT = N_PAD // NS
DEGW = 128
DEG_CH = 128
DEG_CPT = E_PAD // (NW * DEG_CH)
PAD_NODE = N_PAD - 8

_mesh = plsc.VectorSubcoreMesh(core_axis_name="c", subcore_axis_name="s")


@functools.partial(
    pl.kernel,
    out_type=jax.ShapeDtypeStruct((NC, N_PAD, DEGW), jnp.float32),
    mesh=_mesh,
    scratch_types=[
        pltpu.VMEM((DEG_CPT, DEG_CH), jnp.int32),
        pltpu.VMEM((DEG_CH, DEGW), jnp.float32),
        pltpu.VMEM_SHARED((N_PAD, DEGW), jnp.float32),
    ],
)
def _deg_kernel(col_hbm, ones_hbm, zeros_hbm, out_hbm, cidx, ones_v, acc_sh):
    cid = lax.axis_index("c")
    sid = lax.axis_index("s")
    wid = cid * NS + sid
    sl = pl.ds(sid * RPT, RPT)
    pltpu.sync_copy(ones_hbm, ones_v)
    pltpu.sync_copy(col_hbm.at[pl.ds(wid * DEG_CPT, DEG_CPT)], cidx)
    pltpu.sync_copy(zeros_hbm.at[sl], acc_sh.at[sl])
    plsc.subcore_barrier()

    def body(i, carry):
        pltpu.sync_copy(ones_v, acc_sh.at[cidx.at[i]], add=True)
        return carry

    lax.fori_loop(0, DEG_CPT, body, 0)
    plsc.subcore_barrier()
    pltpu.sync_copy(acc_sh.at[sl], out_hbm.at[cid, sl])


@functools.partial(
    pl.kernel,
    out_type=jax.ShapeDtypeStruct((NC, N_PAD, NF), jnp.float32),
    mesh=_mesh,
    scratch_types=[
        pltpu.VMEM((SEG, CH), jnp.int32),
        pltpu.VMEM((SEG, CH), jnp.int32),
        pltpu.VMEM((CH, NF), jnp.float32),
        pltpu.VMEM((CH, NF), jnp.float32),
        pltpu.SemaphoreType.DMA,
        pltpu.SemaphoreType.DMA,
        pltpu.VMEM_SHARED((N_PAD, NF), jnp.float32),
    ],
)
def _agg_kernel(g_hbm, row_hbm, col_hbm, zeros_hbm, out_hbm,
                ridx, cidx, rows0, rows1, sem0, sem1, acc_sh):
    cid = lax.axis_index("c")
    sid = lax.axis_index("s")
    wid = cid * NS + sid
    sl = pl.ds(sid * RPT, RPT)

    @pl.when(cid == 0)
    def _():
        pltpu.sync_copy(g_hbm.at[sl], acc_sh.at[sl])

    @pl.when(cid != 0)
    def _():
        pltpu.sync_copy(zeros_hbm.at[sl], acc_sh.at[sl])

    plsc.subcore_barrier()

    for s in range(NSEG):
        b = wid * CPT + s * SEG
        pltpu.sync_copy(row_hbm.at[pl.ds(b, SEG)], ridx)
        pltpu.sync_copy(col_hbm.at[pl.ds(b, SEG)], cidx)

        def body(m, carry):
            i0 = 2 * m
            c0 = pltpu.async_copy(g_hbm.at[ridx.at[i0]], rows0, sem0)
            c1 = pltpu.async_copy(g_hbm.at[ridx.at[i0 + 1]], rows1, sem1)
            c0.wait()
            pltpu.sync_copy(rows0, acc_sh.at[cidx.at[i0]], add=True)
            c1.wait()
            pltpu.sync_copy(rows1, acc_sh.at[cidx.at[i0 + 1]], add=True)
            return carry

        lax.fori_loop(0, SEG // 2, body, 0)

    plsc.subcore_barrier()
    pltpu.sync_copy(acc_sh.at[sl], out_hbm.at[cid, sl])


BLK = 1000


def _t1_body(x_ref, w_ref, dinv_ref, o_ref):
    h = jnp.dot(x_ref[...], w_ref[...], preferred_element_type=jnp.float32)
    o_ref[...] = dinv_ref[...] * h


_t1 = pl.pallas_call(
    _t1_body,
    grid=(N_NODES // BLK,),
    in_specs=[
        pl.BlockSpec((BLK, NF), lambda i: (i, 0)),
        pl.BlockSpec((NF, NF), lambda i: (0, 0)),
        pl.BlockSpec((BLK, 1), lambda i: (i, 0)),
    ],
    out_specs=pl.BlockSpec((BLK, NF), lambda i: (i, 0)),
    out_shape=jax.ShapeDtypeStruct((N_NODES, NF), jnp.float32),
)


def _t2_body(acc_a_ref, acc_b_ref, dinv_ref, b_ref, w_ref, o_ref):
    dinv = dinv_ref[...]
    h = dinv * (acc_a_ref[...] + acc_b_ref[...]) + b_ref[...]
    h = jnp.maximum(h, 0.0)
    o_ref[...] = dinv * jnp.dot(h, w_ref[...], preferred_element_type=jnp.float32)


_t2 = pl.pallas_call(
    _t2_body,
    grid=(N_NODES // BLK,),
    in_specs=[
        pl.BlockSpec((BLK, NF), lambda i: (i, 0)),
        pl.BlockSpec((BLK, NF), lambda i: (i, 0)),
        pl.BlockSpec((BLK, 1), lambda i: (i, 0)),
        pl.BlockSpec((1, NF), lambda i: (0, 0)),
        pl.BlockSpec((NF, NF), lambda i: (0, 0)),
    ],
    out_specs=pl.BlockSpec((BLK, NF), lambda i: (i, 0)),
    out_shape=jax.ShapeDtypeStruct((N_NODES, NF), jnp.float32),
)


def _t3_body(acc_a_ref, acc_b_ref, dinv_ref, b_ref, wc_ref, bc_ref, o_ref):
    h = dinv_ref[...] * (acc_a_ref[...] + acc_b_ref[...]) + b_ref[...]
    h = jnp.maximum(h, 0.0)
    o_ref[...] = jnp.dot(h, wc_ref[...], preferred_element_type=jnp.float32) + bc_ref[...]


_t3 = pl.pallas_call(
    _t3_body,
    grid=(N_NODES // BLK,),
    in_specs=[
        pl.BlockSpec((BLK, NF), lambda i: (i, 0)),
        pl.BlockSpec((BLK, NF), lambda i: (i, 0)),
        pl.BlockSpec((BLK, 1), lambda i: (i, 0)),
        pl.BlockSpec((1, NF), lambda i: (0, 0)),
        pl.BlockSpec((NF, NCLS), lambda i: (0, 0)),
        pl.BlockSpec((1, NCLS), lambda i: (0, 0)),
    ],
    out_specs=pl.BlockSpec((BLK, NCLS), lambda i: (i, 0)),
    out_shape=jax.ShapeDtypeStruct((N_NODES, NCLS), jnp.float32),
)


def _pad_nodes(a):
    return jnp.pad(a, ((0, N_PAD - N_NODES), (0, 0)))


def kernel(x, edge_index, W1, b1, W2, b2, Wc, bc):
    pad_tgt = jnp.tile(jnp.arange(N_NODES, N_PAD, dtype=jnp.int32),
                       (E_PAD - N_EDGES) // (N_PAD - N_NODES))
    row = jnp.concatenate([edge_index[0].astype(jnp.int32), pad_tgt])
    col = jnp.concatenate([edge_index[1].astype(jnp.int32), pad_tgt])
    row2d = row.reshape(NW * DEG_CPT, DEG_CH)
    col2d = col.reshape(NW * DEG_CPT, DEG_CH)
    ones16 = jnp.ones((DEG_CH, DEGW), jnp.float32)
    zeros16 = jnp.zeros((N_PAD, DEGW), jnp.float32)
    zerosf = jnp.zeros((N_PAD, NF), jnp.float32)

    deg_parts = _deg_kernel(col2d, ones16, zeros16)
    deg = deg_parts[0, :N_NODES, 0] + deg_parts[1, :N_NODES, 0] + 1.0
    dinv = jax.lax.rsqrt(deg).reshape(N_NODES, 1)

    g1 = _pad_nodes(_t1(x, W1, dinv))
    acc1 = _agg_kernel(g1, row2d, col2d, zerosf)
    g2 = _pad_nodes(
        _t2(acc1[0, :N_NODES], acc1[1, :N_NODES], dinv, b1.reshape(1, NF), W2))
    acc2 = _agg_kernel(g2, row2d, col2d, zerosf)
    out = _t3(acc2[0, :N_NODES], acc2[1, :N_NODES], dinv,
              b2.reshape(1, NF), Wc, bc.reshape(1, NCLS))
    return out


# speedup vs baseline: 3.4080x; 1.1152x over previous
import functools

import jax
import jax.numpy as jnp
from jax import lax
from jax.experimental import pallas as pl
from jax.experimental.pallas import tpu as pltpu
from jax.experimental.pallas import tpu_sc as plsc

N_NODES = 10000
N_PAD = 10240
N_EDGES = 320000
NF = 128
NCLS = 40

NC, NS = 2, 16
NW = NC * NS
CH = 128
CPT = 80
SEG = 40
NSEG = CPT // SEG
E_PAD = NW * CPT * CH
RPT = N_PAD // NS
DEGW = 16
DEG_CH = 128
DEG_CPT = E_PAD // (NW * DEG_CH)
PAD_NODE = N_PAD - 8

_mesh = plsc.VectorSubcoreMesh(core_axis_name="c", subcore_axis_name="s")


@functools.partial(
    pl.kernel,
    out_type=jax.ShapeDtypeStruct((NC, N_PAD, DEGW), jnp.float32),
    mesh=_mesh,
    scratch_types=[
        pltpu.VMEM((DEG_CPT, DEG_CH), jnp.int32),
        pltpu.VMEM((DEG_CH, DEGW), jnp.float32),
        pltpu.VMEM_SHARED((N_PAD, DEGW), jnp.float32),
    ],
    compiler_params=pltpu.CompilerParams(use_tc_tiling_on_sc=False),
)
def _deg_kernel(col_hbm, ones_hbm, zeros_hbm, out_hbm, cidx, ones_v, acc_sh):
    cid = lax.axis_index("c")
    sid = lax.axis_index("s")
    wid = cid * NS + sid
    sl = pl.ds(sid * RPT, RPT)
    pltpu.sync_copy(ones_hbm, ones_v)
    pltpu.sync_copy(col_hbm.at[pl.ds(wid * DEG_CPT, DEG_CPT)], cidx)
    pltpu.sync_copy(zeros_hbm.at[sl], acc_sh.at[sl])
    plsc.subcore_barrier()

    def body(i, carry):
        pltpu.sync_copy(ones_v, acc_sh.at[cidx.at[i]], add=True)
        return carry

    lax.fori_loop(0, DEG_CPT, body, 0)
    plsc.subcore_barrier()
    pltpu.sync_copy(acc_sh.at[sl], out_hbm.at[cid, sl])


@functools.partial(
    pl.kernel,
    out_type=jax.ShapeDtypeStruct((NC, N_PAD, NF), jnp.float32),
    mesh=_mesh,
    scratch_types=[
        pltpu.VMEM((SEG, CH), jnp.int32),
        pltpu.VMEM((SEG, CH), jnp.int32),
        pltpu.VMEM((CH, NF), jnp.float32),
        pltpu.VMEM((CH, NF), jnp.float32),
        pltpu.SemaphoreType.DMA,
        pltpu.SemaphoreType.DMA,
        pltpu.VMEM_SHARED((N_PAD, NF), jnp.float32),
    ],
)
def _agg_kernel(g_hbm, row_hbm, col_hbm, zeros_hbm, out_hbm,
                ridx, cidx, rows0, rows1, sem0, sem1, acc_sh):
    cid = lax.axis_index("c")
    sid = lax.axis_index("s")
    wid = cid * NS + sid
    sl = pl.ds(sid * RPT, RPT)

    @pl.when(cid == 0)
    def _():
        pltpu.sync_copy(g_hbm.at[sl], acc_sh.at[sl])

    @pl.when(cid != 0)
    def _():
        pltpu.sync_copy(zeros_hbm.at[sl], acc_sh.at[sl])

    plsc.subcore_barrier()

    for s in range(NSEG):
        b = wid * CPT + s * SEG
        pltpu.sync_copy(row_hbm.at[pl.ds(b, SEG)], ridx)
        pltpu.sync_copy(col_hbm.at[pl.ds(b, SEG)], cidx)

        def body(m, carry):
            i0 = 2 * m
            c0 = pltpu.async_copy(g_hbm.at[ridx.at[i0]], rows0, sem0)
            c1 = pltpu.async_copy(g_hbm.at[ridx.at[i0 + 1]], rows1, sem1)
            c0.wait()
            pltpu.sync_copy(rows0, acc_sh.at[cidx.at[i0]], add=True)
            c1.wait()
            pltpu.sync_copy(rows1, acc_sh.at[cidx.at[i0 + 1]], add=True)
            return carry

        lax.fori_loop(0, SEG // 2, body, 0)

    plsc.subcore_barrier()
    pltpu.sync_copy(acc_sh.at[sl], out_hbm.at[cid, sl])


BLK = 1000


def _t1_body(x_ref, w_ref, dinv_ref, o_ref):
    h = jnp.dot(x_ref[...], w_ref[...], preferred_element_type=jnp.float32)
    o_ref[...] = dinv_ref[...] * h


_t1 = pl.pallas_call(
    _t1_body,
    grid=(N_NODES // BLK,),
    in_specs=[
        pl.BlockSpec((BLK, NF), lambda i: (i, 0)),
        pl.BlockSpec((NF, NF), lambda i: (0, 0)),
        pl.BlockSpec((BLK, 1), lambda i: (i, 0)),
    ],
    out_specs=pl.BlockSpec((BLK, NF), lambda i: (i, 0)),
    out_shape=jax.ShapeDtypeStruct((N_NODES, NF), jnp.float32),
)


def _t2_body(acc_a_ref, acc_b_ref, dinv_ref, b_ref, w_ref, o_ref):
    dinv = dinv_ref[...]
    h = dinv * (acc_a_ref[...] + acc_b_ref[...]) + b_ref[...]
    h = jnp.maximum(h, 0.0)
    o_ref[...] = dinv * jnp.dot(h, w_ref[...], preferred_element_type=jnp.float32)


_t2 = pl.pallas_call(
    _t2_body,
    grid=(N_NODES // BLK,),
    in_specs=[
        pl.BlockSpec((BLK, NF), lambda i: (i, 0)),
        pl.BlockSpec((BLK, NF), lambda i: (i, 0)),
        pl.BlockSpec((BLK, 1), lambda i: (i, 0)),
        pl.BlockSpec((1, NF), lambda i: (0, 0)),
        pl.BlockSpec((NF, NF), lambda i: (0, 0)),
    ],
    out_specs=pl.BlockSpec((BLK, NF), lambda i: (i, 0)),
    out_shape=jax.ShapeDtypeStruct((N_NODES, NF), jnp.float32),
)


def _t3_body(acc_a_ref, acc_b_ref, dinv_ref, b_ref, wc_ref, bc_ref, o_ref):
    h = dinv_ref[...] * (acc_a_ref[...] + acc_b_ref[...]) + b_ref[...]
    h = jnp.maximum(h, 0.0)
    o_ref[...] = jnp.dot(h, wc_ref[...], preferred_element_type=jnp.float32) + bc_ref[...]


_t3 = pl.pallas_call(
    _t3_body,
    grid=(N_NODES // BLK,),
    in_specs=[
        pl.BlockSpec((BLK, NF), lambda i: (i, 0)),
        pl.BlockSpec((BLK, NF), lambda i: (i, 0)),
        pl.BlockSpec((BLK, 1), lambda i: (i, 0)),
        pl.BlockSpec((1, NF), lambda i: (0, 0)),
        pl.BlockSpec((NF, NCLS), lambda i: (0, 0)),
        pl.BlockSpec((1, NCLS), lambda i: (0, 0)),
    ],
    out_specs=pl.BlockSpec((BLK, NCLS), lambda i: (i, 0)),
    out_shape=jax.ShapeDtypeStruct((N_NODES, NCLS), jnp.float32),
)


def _pad_nodes(a):
    return jnp.pad(a, ((0, N_PAD - N_NODES), (0, 0)))


def kernel(x, edge_index, W1, b1, W2, b2, Wc, bc):
    pad_tgt = jnp.tile(jnp.arange(N_NODES, N_PAD, dtype=jnp.int32),
                       (E_PAD - N_EDGES) // (N_PAD - N_NODES))
    row = jnp.concatenate([edge_index[0].astype(jnp.int32), pad_tgt])
    col = jnp.concatenate([edge_index[1].astype(jnp.int32), pad_tgt])
    row2d = row.reshape(NW * DEG_CPT, DEG_CH)
    col2d = col.reshape(NW * DEG_CPT, DEG_CH)
    ones16 = jnp.ones((DEG_CH, DEGW), jnp.float32)
    zeros16 = jnp.zeros((N_PAD, DEGW), jnp.float32)
    zerosf = jnp.zeros((N_PAD, NF), jnp.float32)

    deg_parts = _deg_kernel(col2d, ones16, zeros16)
    deg = deg_parts[0, :N_NODES, 0] + deg_parts[1, :N_NODES, 0] + 1.0
    dinv = jax.lax.rsqrt(deg).reshape(N_NODES, 1)

    g1 = _pad_nodes(_t1(x, W1, dinv))
    acc1 = _agg_kernel(g1, row2d, col2d, zerosf)
    g2 = _pad_nodes(
        _t2(acc1[0, :N_NODES], acc1[1, :N_NODES], dinv, b1.reshape(1, NF), W2))
    acc2 = _agg_kernel(g2, row2d, col2d, zerosf)
    out = _t3(acc2[0, :N_NODES], acc2[1, :N_NODES], dinv,
              b2.reshape(1, NF), Wc, bc.reshape(1, NCLS))
    return out
